# Initial kernel scaffold; baseline (speedup 1.0000x reference)
import jax
import jax.numpy as jnp
from jax.experimental import pallas as pl


def kernel(data, embed_weight):
    raise NotImplementedError("write your pallas kernel here")


# speedup vs baseline: 136.3826x; 136.3826x over previous
import jax
import jax.numpy as jnp
from jax.experimental import pallas as pl

_BATCH = 1024
_CLASSES = 1000
_DIM = 16
_OUT_W = _CLASSES * _DIM
_ROWS = 128


def _fill_body(data_ref, t0_ref, t1_ref, out_ref):
    d = data_ref[...]
    col_block = jax.lax.broadcasted_iota(jnp.int32, (1, _OUT_W), 1) >> 4
    mask = col_block == d
    out_ref[...] = jnp.where(mask, t1_ref[...], t0_ref[...])


def kernel(data, embed_weight):
    t0 = jnp.broadcast_to(embed_weight[0:1, :], (_CLASSES, _DIM)).reshape(1, _OUT_W)
    t1 = jnp.broadcast_to(embed_weight[1:2, :], (_CLASSES, _DIM)).reshape(1, _OUT_W)
    data2 = data.reshape(_BATCH, 1)
    grid = (_BATCH // _ROWS,)
    out = pl.pallas_call(
        _fill_body,
        grid=grid,
        in_specs=[
            pl.BlockSpec((_ROWS, 1), lambda i: (i, 0)),
            pl.BlockSpec((1, _OUT_W), lambda i: (0, 0)),
            pl.BlockSpec((1, _OUT_W), lambda i: (0, 0)),
        ],
        out_specs=pl.BlockSpec((_ROWS, _OUT_W), lambda i: (i, 0)),
        out_shape=jax.ShapeDtypeStruct((_BATCH, _OUT_W), jnp.float32),
    )(data2, t0, t1)
    return out
